# baseline (speedup 1.0000x reference)
import functools

import jax
import jax.numpy as jnp
import numpy as np
from jax import lax
from jax.experimental import pallas as pl
from jax.experimental.pallas import tpu as pltpu
from jax.experimental.pallas import tpu_sc as plsc

N_NODES = 50000
N_EDGES = 800000
N_HIDDEN = 64
N_GRAPHS = 64

NC = 2
NS = 16
NW = NC * NS

_CH = 128


_G_EPW = N_EDGES // NW
_G_FULL = _G_EPW // _CH
_G_TAIL = _G_EPW - _G_FULL * _CH


def _sc_gather2(hA, hB, row, col):
    H2 = 2 * N_HIDDEN
    mesh = plsc.VectorSubcoreMesh(
        core_axis_name="c", subcore_axis_name="s", num_cores=NC, num_subcores=NS)

    @functools.partial(
        pl.kernel,
        out_type=(jax.ShapeDtypeStruct((N_EDGES, H2), jnp.float32),
                  jax.ShapeDtypeStruct((N_EDGES, H2), jnp.float32)),
        mesh=mesh,
        scratch_types=[
            pltpu.VMEM((_CH,), jnp.int32),
            pltpu.VMEM((_CH,), jnp.int32),
            pltpu.VMEM((_CH, H2), jnp.float32),
            pltpu.VMEM((_CH, H2), jnp.float32),
            pltpu.VMEM((_G_TAIL,), jnp.int32),
            pltpu.VMEM((_G_TAIL,), jnp.int32),
            pltpu.VMEM((_G_TAIL, H2), jnp.float32),
            pltpu.VMEM((_G_TAIL, H2), jnp.float32),
            pltpu.SemaphoreType.DMA,
            pltpu.SemaphoreType.DMA,
        ],
    )
    def k(hA_hbm, hB_hbm, row_hbm, col_hbm, outA_hbm, outB_hbm,
          ir, ic, bufA, bufB, irt, ict, bufAt, bufBt, s1, s2):
        wid = lax.axis_index("s") * NC + lax.axis_index("c")
        base0 = wid * _G_EPW

        def body(j, carry):
            base = base0 + j * _CH
            pltpu.sync_copy(row_hbm.at[pl.ds(base, _CH)], ir)
            pltpu.sync_copy(col_hbm.at[pl.ds(base, _CH)], ic)
            d1 = pltpu.async_copy(hA_hbm.at[ir], bufA, s1)
            d2 = pltpu.async_copy(hB_hbm.at[ic], bufB, s2)
            d1.wait()
            d2.wait()
            pltpu.sync_copy(bufA, outA_hbm.at[pl.ds(base, _CH)])
            pltpu.sync_copy(bufB, outB_hbm.at[pl.ds(base, _CH)])
            return carry

        lax.fori_loop(0, _G_FULL, body, 0)
        base = base0 + _G_FULL * _CH
        pltpu.sync_copy(row_hbm.at[pl.ds(base, _G_TAIL)], irt)
        pltpu.sync_copy(col_hbm.at[pl.ds(base, _G_TAIL)], ict)
        d1 = pltpu.async_copy(hA_hbm.at[irt], bufAt, s1)
        d2 = pltpu.async_copy(hB_hbm.at[ict], bufBt, s2)
        d1.wait()
        d2.wait()
        pltpu.sync_copy(bufAt, outA_hbm.at[pl.ds(base, _G_TAIL)])
        pltpu.sync_copy(bufBt, outB_hbm.at[pl.ds(base, _G_TAIL)])

    return k(hA, hB, row, col)


_NPAD = 50048
_FPW = N_HIDDEN // NW
_S_CHUNKS = N_EDGES // _CH


def _sc_scatter(eoT, col, zeros1d):
    mesh = plsc.VectorSubcoreMesh(
        core_axis_name="c", subcore_axis_name="s", num_cores=NC, num_subcores=NS)

    @functools.partial(
        pl.kernel,
        out_type=jax.ShapeDtypeStruct((N_HIDDEN, _NPAD), jnp.float32),
        mesh=mesh,
        compiler_params=pltpu.CompilerParams(needs_layout_passes=False),
        scratch_types=[
            pltpu.VMEM((_FPW * _NPAD,), jnp.float32),
            pltpu.VMEM((_CH,), jnp.int32),
            pltpu.VMEM((_CH,), jnp.float32),
            pltpu.VMEM((_CH,), jnp.float32),
        ],
    )
    def k(eoT_hbm, col_hbm, zeros_hbm, out_hbm, tbl, iv, v0, v1):
        wid = lax.axis_index("s") * NC + lax.axis_index("c")
        f0 = wid * _FPW
        pltpu.sync_copy(zeros_hbm, tbl)

        def body(j, carry):
            base = j * _CH
            pltpu.sync_copy(col_hbm.at[pl.ds(base, _CH)], iv)
            pltpu.sync_copy(eoT_hbm.at[f0, pl.ds(base, _CH)], v0)
            pltpu.sync_copy(eoT_hbm.at[f0 + 1, pl.ds(base, _CH)], v1)
            for t in range(_CH // 16):
                sl = pl.ds(t * 16, 16)
                idx = iv[sl]
                plsc.addupdate_scatter(tbl, [idx], v0[sl])
                plsc.addupdate_scatter(tbl, [idx + _NPAD], v1[sl])
            return carry

        lax.fori_loop(0, _S_CHUNKS, body, 0)

        pltpu.sync_copy(tbl.at[pl.ds(0, _NPAD)], out_hbm.at[f0])
        pltpu.sync_copy(tbl.at[pl.ds(_NPAD, _NPAD)], out_hbm.at[f0 + 1])

    return k(eoT, col, zeros1d)


_D_EPW = N_EDGES // NW
_D_FULL = _D_EPW // _CH
_D_TAIL = _D_EPW - _D_FULL * _CH


def _sc_degree(col, zeros1d):
    mesh = plsc.VectorSubcoreMesh(
        core_axis_name="c", subcore_axis_name="s", num_cores=NC, num_subcores=NS)

    @functools.partial(
        pl.kernel,
        out_type=jax.ShapeDtypeStruct((NW, _NPAD), jnp.float32),
        mesh=mesh,
        compiler_params=pltpu.CompilerParams(needs_layout_passes=False),
        scratch_types=[
            pltpu.VMEM((_NPAD,), jnp.float32),
            pltpu.VMEM((_CH,), jnp.int32),
            pltpu.VMEM((_D_TAIL,), jnp.int32),
        ],
    )
    def k(col_hbm, zeros_hbm, out_hbm, tbl, iv, ivt):
        wid = lax.axis_index("s") * NC + lax.axis_index("c")
        base0 = wid * _D_EPW
        pltpu.sync_copy(zeros_hbm, tbl)
        ones16 = jnp.full((16,), 1.0, jnp.float32)

        def body(j, carry):
            base = base0 + j * _CH
            pltpu.sync_copy(col_hbm.at[pl.ds(base, _CH)], iv)
            for t in range(_CH // 16):
                plsc.addupdate_scatter(tbl, [iv[pl.ds(t * 16, 16)]], ones16)
            return carry

        lax.fori_loop(0, _D_FULL, body, 0)
        base = base0 + _D_FULL * _CH
        pltpu.sync_copy(col_hbm.at[pl.ds(base, _D_TAIL)], ivt)
        for t in range(_D_TAIL // 16):
            plsc.addupdate_scatter(tbl, [ivt[pl.ds(t * 16, 16)]], ones16)
        rem = _D_TAIL - (_D_TAIL // 16) * 16
        if rem:
            idx = ivt[pl.ds(_D_TAIL - 16, 16)]
            tailmask = lax.iota(jnp.int32, 16) >= (16 - rem)
            plsc.addupdate_scatter(tbl, [idx], ones16, mask=tailmask)

        pltpu.sync_copy(tbl, out_hbm.at[wid])

    return k(col, zeros1d)


_NB = 2000
_EB = 6400


def _enc_nodes_body(x_ref, W1_ref, b1_ref, a_ref, W2_ref, b2_ref, o_ref):
    u = jnp.dot(x_ref[...], W1_ref[...], preferred_element_type=jnp.float32)
    u = u + b1_ref[...]
    u = jnp.where(u >= 0, u, a_ref[...] * u)
    o_ref[...] = (jnp.dot(u, W2_ref[...], preferred_element_type=jnp.float32)
                  + b2_ref[...])


def _enc_nodes(x, W1, b1, a, W2, b2):
    n_atom = x.shape[1]
    return pl.pallas_call(
        _enc_nodes_body,
        grid=(N_NODES // _NB,),
        in_specs=[
            pl.BlockSpec((_NB, n_atom), lambda i: (i, 0)),
            pl.BlockSpec((n_atom, N_HIDDEN), lambda i: (0, 0)),
            pl.BlockSpec((1, N_HIDDEN), lambda i: (0, 0)),
            pl.BlockSpec((1, 1), lambda i: (0, 0)),
            pl.BlockSpec((N_HIDDEN, N_HIDDEN), lambda i: (0, 0)),
            pl.BlockSpec((1, N_HIDDEN), lambda i: (0, 0)),
        ],
        out_specs=pl.BlockSpec((_NB, N_HIDDEN), lambda i: (i, 0)),
        out_shape=jax.ShapeDtypeStruct((N_NODES, N_HIDDEN), jnp.float32),
    )(x, W1, b1.reshape(1, -1), a.reshape(1, 1), W2, b2.reshape(1, -1))


def _enc_edges_body(ev_ref, W1_ref, b1_ref, a_ref, W2_ref, b2_ref, o_ref):
    v = ev_ref[...]
    vx = v[:, 0:1]
    vy = v[:, 1:2]
    vz = v[:, 2:3]
    n2 = vx * vx + vy * vy + vz * vz
    norm = jnp.sqrt(n2)
    inv = 1.0 / norm
    s3 = np.float32(np.sqrt(3.0))
    u = 2.0 * (norm * 0.25 - 1.0)
    y = (1.0 - jnp.cos(np.float32(np.pi) * u)) * 0.5
    y = jnp.where(u > 0, 0.0, y)
    y = jnp.where(u < -1, 1.0, y)
    a0 = y
    a1 = y * s3 * vy * inv
    a2 = y * s3 * vz * inv
    a3 = y * s3 * vx * inv
    W1 = W1_ref[...]
    e1 = (a0 * W1[0:1, :] + a1 * W1[1:2, :] + a2 * W1[2:3, :] + a3 * W1[3:4, :]
          + b1_ref[...])
    e1 = jnp.where(e1 >= 0, e1, a_ref[...] * e1)
    o_ref[...] = (jnp.dot(e1, W2_ref[...], preferred_element_type=jnp.float32)
                  + b2_ref[...])


def _enc_edges(edge_vec, W1, b1, a, W2, b2):
    return pl.pallas_call(
        _enc_edges_body,
        grid=(N_EDGES // _EB,),
        in_specs=[
            pl.BlockSpec((_EB, 3), lambda i: (i, 0)),
            pl.BlockSpec((4, N_HIDDEN), lambda i: (0, 0)),
            pl.BlockSpec((1, N_HIDDEN), lambda i: (0, 0)),
            pl.BlockSpec((1, 1), lambda i: (0, 0)),
            pl.BlockSpec((N_HIDDEN, N_HIDDEN), lambda i: (0, 0)),
            pl.BlockSpec((1, N_HIDDEN), lambda i: (0, 0)),
        ],
        out_specs=pl.BlockSpec((_EB, N_HIDDEN), lambda i: (i, 0)),
        out_shape=jax.ShapeDtypeStruct((N_EDGES, N_HIDDEN), jnp.float32),
    )(edge_vec, W1, b1.reshape(1, -1), a.reshape(1, 1), W2, b2.reshape(1, -1))


def _pre_edge_body(h_ref, Wr_ref, Wc_ref, hA_ref, hB_ref):
    hb = h_ref[...]
    hA_ref[...] = jnp.dot(hb, Wr_ref[...], preferred_element_type=jnp.float32)
    hB_ref[...] = jnp.dot(hb, Wc_ref[...], preferred_element_type=jnp.float32)


def _pre_edge(h, Wr, Wc):
    H2 = 2 * N_HIDDEN
    return pl.pallas_call(
        _pre_edge_body,
        grid=(N_NODES // _NB,),
        in_specs=[
            pl.BlockSpec((_NB, N_HIDDEN), lambda i: (i, 0)),
            pl.BlockSpec((N_HIDDEN, H2), lambda i: (0, 0)),
            pl.BlockSpec((N_HIDDEN, H2), lambda i: (0, 0)),
        ],
        out_specs=[
            pl.BlockSpec((_NB, H2), lambda i: (i, 0)),
            pl.BlockSpec((_NB, H2), lambda i: (i, 0)),
        ],
        out_shape=[
            jax.ShapeDtypeStruct((N_NODES, H2), jnp.float32),
            jax.ShapeDtypeStruct((N_NODES, H2), jnp.float32),
        ],
    )(h, Wr, Wc)


def _ln_prelu(u, g_ref, be_ref, a_ref):
    m = jnp.mean(u, axis=1, keepdims=True)
    d = u - m
    v = jnp.mean(d * d, axis=1, keepdims=True)
    u = d * jax.lax.rsqrt(v + 1e-5) * g_ref[...] + be_ref[...]
    return jnp.where(u >= 0, u, a_ref[...] * u)


def _edge_mlp_body(epA_ref, epB_ref, e_ref, We_ref, b1_ref, g_ref, be_ref,
                   a_ref, W2_ref, b2_ref, eoT_ref, en_ref):
    u = (epA_ref[...] + epB_ref[...]
         + jnp.dot(e_ref[...], We_ref[...], preferred_element_type=jnp.float32)
         + b1_ref[...])
    u = _ln_prelu(u, g_ref, be_ref, a_ref)
    eo = (jnp.dot(u, W2_ref[...], preferred_element_type=jnp.float32)
          + b2_ref[...])
    eoT_ref[...] = eo.T
    en_ref[...] = e_ref[...] + eo


def _edge_mlp(epA, epB, e, We, b1, g, be, a, W2, b2):
    H2 = 2 * N_HIDDEN
    wide = pl.BlockSpec((1, H2), lambda i: (0, 0))
    return pl.pallas_call(
        _edge_mlp_body,
        grid=(N_EDGES // _EB,),
        in_specs=[
            pl.BlockSpec((_EB, H2), lambda i: (i, 0)),
            pl.BlockSpec((_EB, H2), lambda i: (i, 0)),
            pl.BlockSpec((_EB, N_HIDDEN), lambda i: (i, 0)),
            pl.BlockSpec((N_HIDDEN, H2), lambda i: (0, 0)),
            wide, wide, wide,
            pl.BlockSpec((1, 1), lambda i: (0, 0)),
            pl.BlockSpec((H2, N_HIDDEN), lambda i: (0, 0)),
            pl.BlockSpec((1, N_HIDDEN), lambda i: (0, 0)),
        ],
        out_specs=[
            pl.BlockSpec((N_HIDDEN, _EB), lambda i: (0, i)),
            pl.BlockSpec((_EB, N_HIDDEN), lambda i: (i, 0)),
        ],
        out_shape=[
            jax.ShapeDtypeStruct((N_HIDDEN, N_EDGES), jnp.float32),
            jax.ShapeDtypeStruct((N_EDGES, N_HIDDEN), jnp.float32),
        ],
    )(epA, epB, e, We, b1.reshape(1, -1), g.reshape(1, -1),
      be.reshape(1, -1), a.reshape(1, 1), W2, b2.reshape(1, -1))


def _agg_mean_body(aggT_ref, degp_ref, o_ref):
    deg = jnp.maximum(jnp.sum(degp_ref[...], axis=0, keepdims=True), 1.0)
    o_ref[...] = (aggT_ref[...] / deg).T


_AB = 2176


def _agg_mean(aggT, degp):
    return pl.pallas_call(
        _agg_mean_body,
        grid=(_NPAD // _AB,),
        in_specs=[
            pl.BlockSpec((N_HIDDEN, _AB), lambda i: (0, i)),
            pl.BlockSpec((NW, _AB), lambda i: (0, i)),
        ],
        out_specs=pl.BlockSpec((_AB, N_HIDDEN), lambda i: (i, 0)),
        out_shape=jax.ShapeDtypeStruct((_NPAD, N_HIDDEN), jnp.float32),
    )(aggT, degp)


def _node_mlp_body(h_ref, agg_ref, Wh_ref, Wa_ref, b1_ref,
                   g_ref, be_ref, a_ref, W2_ref, b2_ref, o_ref):
    hb = h_ref[...]
    u = (jnp.dot(hb, Wh_ref[...], preferred_element_type=jnp.float32)
         + jnp.dot(agg_ref[...], Wa_ref[...],
                   preferred_element_type=jnp.float32)
         + b1_ref[...])
    u = _ln_prelu(u, g_ref, be_ref, a_ref)
    no = (jnp.dot(u, W2_ref[...], preferred_element_type=jnp.float32)
          + b2_ref[...])
    o_ref[...] = hb + no


def _node_mlp(h, agg, Wh, Wa, b1, g, be, a, W2, b2):
    H2 = 2 * N_HIDDEN
    wide = pl.BlockSpec((1, H2), lambda i: (0, 0))
    return pl.pallas_call(
        _node_mlp_body,
        grid=(N_NODES // _NB,),
        in_specs=[
            pl.BlockSpec((_NB, N_HIDDEN), lambda i: (i, 0)),
            pl.BlockSpec((_NB, N_HIDDEN), lambda i: (i, 0)),
            pl.BlockSpec((N_HIDDEN, H2), lambda i: (0, 0)),
            pl.BlockSpec((N_HIDDEN, H2), lambda i: (0, 0)),
            wide, wide, wide,
            pl.BlockSpec((1, 1), lambda i: (0, 0)),
            pl.BlockSpec((H2, N_HIDDEN), lambda i: (0, 0)),
            pl.BlockSpec((1, N_HIDDEN), lambda i: (0, 0)),
        ],
        out_specs=pl.BlockSpec((_NB, N_HIDDEN), lambda i: (i, 0)),
        out_shape=jax.ShapeDtypeStruct((N_NODES, N_HIDDEN), jnp.float32),
    )(h, agg, Wh, Wa, b1.reshape(1, -1), g.reshape(1, -1),
      be.reshape(1, -1), a.reshape(1, 1), W2, b2.reshape(1, -1))


_OB = 2000


def _out_body(h_ref, b3_ref, W1_ref, b1_ref, W2_ref, b2_ref, o_ref, acc_ref):
    i = pl.program_id(0)

    @pl.when(i == 0)
    def _():
        acc_ref[...] = jnp.zeros_like(acc_ref)

    gids = lax.broadcasted_iota(jnp.int32, (N_GRAPHS, _OB), 0)
    onehot = (gids == b3_ref[0]).astype(jnp.float32)
    acc_ref[...] += jnp.dot(onehot, h_ref[...],
                            preferred_element_type=jnp.float32)

    @pl.when(i == (N_NODES // _OB) - 1)
    def _():
        hid = (jnp.dot(acc_ref[...], W1_ref[...],
                       preferred_element_type=jnp.float32) + b1_ref[...])
        hid = jnp.where(hid >= 0, hid, 0.01 * hid)
        o_ref[...] = (jnp.dot(hid, W2_ref[...],
                              preferred_element_type=jnp.float32) + b2_ref[...])


def _out_mlp(h, batch3d, W1, b1, W2, b2):
    nh = W1.shape[1]
    nout = W2.shape[1]
    return pl.pallas_call(
        _out_body,
        grid=(N_NODES // _OB,),
        in_specs=[
            pl.BlockSpec((_OB, N_HIDDEN), lambda i: (i, 0)),
            pl.BlockSpec((1, 1, _OB), lambda i: (i, 0, 0)),
            pl.BlockSpec((N_HIDDEN, nh), lambda i: (0, 0)),
            pl.BlockSpec((1, nh), lambda i: (0, 0)),
            pl.BlockSpec((nh, nout), lambda i: (0, 0)),
            pl.BlockSpec((1, nout), lambda i: (0, 0)),
        ],
        out_specs=pl.BlockSpec((N_GRAPHS, nout), lambda i: (0, 0)),
        out_shape=jax.ShapeDtypeStruct((N_GRAPHS, nout), jnp.float32),
        scratch_shapes=[pltpu.VMEM((N_GRAPHS, N_HIDDEN), jnp.float32)],
    )(h, batch3d, W1, b1.reshape(1, -1), W2, b2.reshape(1, -1))


def kernel(x, edge_vec, edge_index, batch, params):
    p = params
    row = edge_index[0]
    col = edge_index[1]

    h = _enc_nodes(x, p['enc_n_W1'], p['enc_n_b1'], p['enc_n_a'],
                   p['enc_n_W2'], p['enc_n_b2'])
    e = _enc_edges(edge_vec, p['enc_e_W1'], p['enc_e_b1'], p['enc_e_a'],
                   p['enc_e_W2'], p['enc_e_b2'])

    def _fill(shape, val):
        return pl.pallas_call(
            lambda x_ref, o_ref: o_ref.__setitem__(
                (Ellipsis,), jnp.full(o_ref.shape, val, jnp.float32)
                + x_ref[0, 0] * 0.0),
            out_shape=jax.ShapeDtypeStruct(shape, jnp.float32),
        )(x)

    zeros_deg = _fill((_NPAD // 128, 128), 0.0).reshape(_NPAD)
    zeros_sc = _fill((_FPW * _NPAD // 128, 128), 0.0).reshape(_FPW * _NPAD)
    degp = _sc_degree(col, zeros_deg)

    for lp in p['layers']:
        eW1 = lp['e_W1']
        hA, hB = _pre_edge(h, eW1[0:N_HIDDEN], eW1[N_HIDDEN:2 * N_HIDDEN])
        epA, epB = _sc_gather2(hA, hB, row, col)
        eoT, e = _edge_mlp(epA, epB, e, eW1[2 * N_HIDDEN:], lp['e_b1'],
                           lp['e_g'], lp['e_be'], lp['e_a'], lp['e_W2'],
                           lp['e_b2'])
        aggT = _sc_scatter(eoT, col, zeros_sc)
        agg = _agg_mean(aggT, degp)
        nW1 = lp['n_W1']
        h = _node_mlp(h, agg, nW1[0:N_HIDDEN], nW1[N_HIDDEN:],
                      lp['n_b1'], lp['n_g'], lp['n_be'], lp['n_a'],
                      lp['n_W2'], lp['n_b2'])

    batch3d = batch.reshape(N_NODES // _OB, 1, _OB)
    return _out_mlp(h, batch3d, p['out_W1'], p['out_b1'],
                    p['out_W2'], p['out_b2'])


# speedup vs baseline: 1.7581x; 1.7581x over previous
import functools

import jax
import jax.numpy as jnp
import numpy as np
from jax import lax
from jax.experimental import pallas as pl
from jax.experimental.pallas import tpu as pltpu
from jax.experimental.pallas import tpu_sc as plsc

N_NODES = 50000
N_EDGES = 800000
N_HIDDEN = 64
N_GRAPHS = 64

NC = 2
NS = 16
NW = NC * NS

_CH = 128


_G_EPW = N_EDGES // NW
_G_FULL = _G_EPW // _CH
_G_TAIL = _G_EPW - _G_FULL * _CH


def _sc_gather2(hA, hB, row, col):
    H2 = 2 * N_HIDDEN
    mesh = plsc.VectorSubcoreMesh(
        core_axis_name="c", subcore_axis_name="s", num_cores=NC, num_subcores=NS)

    @functools.partial(
        pl.kernel,
        out_type=(jax.ShapeDtypeStruct((N_EDGES, H2), jnp.float32),
                  jax.ShapeDtypeStruct((N_EDGES, H2), jnp.float32)),
        mesh=mesh,
        scratch_types=[
            pltpu.VMEM((_CH,), jnp.int32),
            pltpu.VMEM((_CH,), jnp.int32),
            pltpu.VMEM((_CH, H2), jnp.float32),
            pltpu.VMEM((_CH, H2), jnp.float32),
            pltpu.VMEM((_G_TAIL,), jnp.int32),
            pltpu.VMEM((_G_TAIL,), jnp.int32),
            pltpu.VMEM((_G_TAIL, H2), jnp.float32),
            pltpu.VMEM((_G_TAIL, H2), jnp.float32),
            pltpu.SemaphoreType.DMA,
            pltpu.SemaphoreType.DMA,
        ],
    )
    def k(hA_hbm, hB_hbm, row_hbm, col_hbm, outA_hbm, outB_hbm,
          ir, ic, bufA, bufB, irt, ict, bufAt, bufBt, s1, s2):
        wid = lax.axis_index("s") * NC + lax.axis_index("c")
        base0 = wid * _G_EPW

        def body(j, carry):
            base = base0 + j * _CH
            pltpu.sync_copy(row_hbm.at[pl.ds(base, _CH)], ir)
            pltpu.sync_copy(col_hbm.at[pl.ds(base, _CH)], ic)
            d1 = pltpu.async_copy(hA_hbm.at[ir], bufA, s1)
            d2 = pltpu.async_copy(hB_hbm.at[ic], bufB, s2)
            d1.wait()
            d2.wait()
            pltpu.sync_copy(bufA, outA_hbm.at[pl.ds(base, _CH)])
            pltpu.sync_copy(bufB, outB_hbm.at[pl.ds(base, _CH)])
            return carry

        lax.fori_loop(0, _G_FULL, body, 0)
        base = base0 + _G_FULL * _CH
        pltpu.sync_copy(row_hbm.at[pl.ds(base, _G_TAIL)], irt)
        pltpu.sync_copy(col_hbm.at[pl.ds(base, _G_TAIL)], ict)
        d1 = pltpu.async_copy(hA_hbm.at[irt], bufAt, s1)
        d2 = pltpu.async_copy(hB_hbm.at[ict], bufBt, s2)
        d1.wait()
        d2.wait()
        pltpu.sync_copy(bufAt, outA_hbm.at[pl.ds(base, _G_TAIL)])
        pltpu.sync_copy(bufBt, outB_hbm.at[pl.ds(base, _G_TAIL)])

    return k(hA, hB, row, col)


_NPAD = 50048
_FPW = N_HIDDEN // NW
_SCH = 4000
_S_CHUNKS = N_EDGES // _SCH


def _sc_scatter(eoT, col, zeros1d):
    mesh = plsc.VectorSubcoreMesh(
        core_axis_name="c", subcore_axis_name="s", num_cores=NC, num_subcores=NS)

    @functools.partial(
        pl.kernel,
        out_type=jax.ShapeDtypeStruct((N_HIDDEN * _NPAD,), jnp.float32),
        mesh=mesh,
        compiler_params=pltpu.CompilerParams(needs_layout_passes=False),
        scratch_types=[
            pltpu.VMEM((_FPW * _NPAD,), jnp.float32),
            pltpu.VMEM((_SCH,), jnp.int32),
            pltpu.VMEM((_SCH,), jnp.float32),
            pltpu.VMEM((_SCH,), jnp.float32),
            pltpu.SemaphoreType.DMA,
            pltpu.SemaphoreType.DMA,
            pltpu.SemaphoreType.DMA,
        ],
    )
    def k(eoT_hbm, col_hbm, zeros_hbm, out_hbm, tbl, iv, v0, v1, s0, s1, s2):
        wid = lax.axis_index("s") * NC + lax.axis_index("c")
        f0 = wid * _FPW
        pltpu.sync_copy(zeros_hbm, tbl)

        def body(j, carry):
            base = j * _SCH
            d0 = pltpu.async_copy(col_hbm.at[pl.ds(base, _SCH)], iv, s0)
            d1 = pltpu.async_copy(
                eoT_hbm.at[pl.ds(f0 * N_EDGES + base, _SCH)], v0, s1)
            d2 = pltpu.async_copy(
                eoT_hbm.at[pl.ds((f0 + 1) * N_EDGES + base, _SCH)], v1, s2)
            d0.wait()
            d1.wait()
            d2.wait()

            def group(t, carry2):
                sl = pl.ds(t * 16, 16)
                idx = iv[sl]
                plsc.addupdate_scatter(tbl, [idx], v0[sl])
                plsc.addupdate_scatter(tbl, [idx + _NPAD], v1[sl])
                return carry2

            lax.fori_loop(0, _SCH // 16, group, 0)
            return carry

        lax.fori_loop(0, _S_CHUNKS, body, 0)

        pltpu.sync_copy(tbl, out_hbm.at[pl.ds(f0 * _NPAD, _FPW * _NPAD)])

    return k(eoT.reshape(N_HIDDEN * N_EDGES), col,
             zeros1d).reshape(N_HIDDEN, _NPAD)


_D_EPW = N_EDGES // NW
_D_FULL = _D_EPW // _CH
_D_TAIL = _D_EPW - _D_FULL * _CH


def _sc_degree(col, zeros1d):
    mesh = plsc.VectorSubcoreMesh(
        core_axis_name="c", subcore_axis_name="s", num_cores=NC, num_subcores=NS)

    @functools.partial(
        pl.kernel,
        out_type=jax.ShapeDtypeStruct((NW, _NPAD), jnp.float32),
        mesh=mesh,
        compiler_params=pltpu.CompilerParams(needs_layout_passes=False),
        scratch_types=[
            pltpu.VMEM((_NPAD,), jnp.float32),
            pltpu.VMEM((_CH,), jnp.int32),
            pltpu.VMEM((_D_TAIL,), jnp.int32),
        ],
    )
    def k(col_hbm, zeros_hbm, out_hbm, tbl, iv, ivt):
        wid = lax.axis_index("s") * NC + lax.axis_index("c")
        base0 = wid * _D_EPW
        pltpu.sync_copy(zeros_hbm, tbl)
        ones16 = jnp.full((16,), 1.0, jnp.float32)

        def body(j, carry):
            base = base0 + j * _CH
            pltpu.sync_copy(col_hbm.at[pl.ds(base, _CH)], iv)
            for t in range(_CH // 16):
                plsc.addupdate_scatter(tbl, [iv[pl.ds(t * 16, 16)]], ones16)
            return carry

        lax.fori_loop(0, _D_FULL, body, 0)
        base = base0 + _D_FULL * _CH
        pltpu.sync_copy(col_hbm.at[pl.ds(base, _D_TAIL)], ivt)
        for t in range(_D_TAIL // 16):
            plsc.addupdate_scatter(tbl, [ivt[pl.ds(t * 16, 16)]], ones16)
        rem = _D_TAIL - (_D_TAIL // 16) * 16
        if rem:
            idx = ivt[pl.ds(_D_TAIL - 16, 16)]
            tailmask = lax.iota(jnp.int32, 16) >= (16 - rem)
            plsc.addupdate_scatter(tbl, [idx], ones16, mask=tailmask)

        pltpu.sync_copy(tbl, out_hbm.at[wid])

    return k(col, zeros1d)


_NB = 2000
_EB = 6400


def _enc_nodes_body(x_ref, W1_ref, b1_ref, a_ref, W2_ref, b2_ref, o_ref):
    u = jnp.dot(x_ref[...], W1_ref[...], preferred_element_type=jnp.float32)
    u = u + b1_ref[...]
    u = jnp.where(u >= 0, u, a_ref[...] * u)
    o_ref[...] = (jnp.dot(u, W2_ref[...], preferred_element_type=jnp.float32)
                  + b2_ref[...])


def _enc_nodes(x, W1, b1, a, W2, b2):
    n_atom = x.shape[1]
    return pl.pallas_call(
        _enc_nodes_body,
        grid=(N_NODES // _NB,),
        in_specs=[
            pl.BlockSpec((_NB, n_atom), lambda i: (i, 0)),
            pl.BlockSpec((n_atom, N_HIDDEN), lambda i: (0, 0)),
            pl.BlockSpec((1, N_HIDDEN), lambda i: (0, 0)),
            pl.BlockSpec((1, 1), lambda i: (0, 0)),
            pl.BlockSpec((N_HIDDEN, N_HIDDEN), lambda i: (0, 0)),
            pl.BlockSpec((1, N_HIDDEN), lambda i: (0, 0)),
        ],
        out_specs=pl.BlockSpec((_NB, N_HIDDEN), lambda i: (i, 0)),
        out_shape=jax.ShapeDtypeStruct((N_NODES, N_HIDDEN), jnp.float32),
    )(x, W1, b1.reshape(1, -1), a.reshape(1, 1), W2, b2.reshape(1, -1))


def _enc_edges_body(ev_ref, W1_ref, b1_ref, a_ref, W2_ref, b2_ref, o_ref):
    v = ev_ref[...]
    vx = v[:, 0:1]
    vy = v[:, 1:2]
    vz = v[:, 2:3]
    n2 = vx * vx + vy * vy + vz * vz
    norm = jnp.sqrt(n2)
    inv = 1.0 / norm
    s3 = np.float32(np.sqrt(3.0))
    u = 2.0 * (norm * 0.25 - 1.0)
    y = (1.0 - jnp.cos(np.float32(np.pi) * u)) * 0.5
    y = jnp.where(u > 0, 0.0, y)
    y = jnp.where(u < -1, 1.0, y)
    a0 = y
    a1 = y * s3 * vy * inv
    a2 = y * s3 * vz * inv
    a3 = y * s3 * vx * inv
    W1 = W1_ref[...]
    e1 = (a0 * W1[0:1, :] + a1 * W1[1:2, :] + a2 * W1[2:3, :] + a3 * W1[3:4, :]
          + b1_ref[...])
    e1 = jnp.where(e1 >= 0, e1, a_ref[...] * e1)
    o_ref[...] = (jnp.dot(e1, W2_ref[...], preferred_element_type=jnp.float32)
                  + b2_ref[...])


def _enc_edges(edge_vec, W1, b1, a, W2, b2):
    return pl.pallas_call(
        _enc_edges_body,
        grid=(N_EDGES // _EB,),
        in_specs=[
            pl.BlockSpec((_EB, 3), lambda i: (i, 0)),
            pl.BlockSpec((4, N_HIDDEN), lambda i: (0, 0)),
            pl.BlockSpec((1, N_HIDDEN), lambda i: (0, 0)),
            pl.BlockSpec((1, 1), lambda i: (0, 0)),
            pl.BlockSpec((N_HIDDEN, N_HIDDEN), lambda i: (0, 0)),
            pl.BlockSpec((1, N_HIDDEN), lambda i: (0, 0)),
        ],
        out_specs=pl.BlockSpec((_EB, N_HIDDEN), lambda i: (i, 0)),
        out_shape=jax.ShapeDtypeStruct((N_EDGES, N_HIDDEN), jnp.float32),
    )(edge_vec, W1, b1.reshape(1, -1), a.reshape(1, 1), W2, b2.reshape(1, -1))


def _pre_edge_body(h_ref, Wr_ref, Wc_ref, hA_ref, hB_ref):
    hb = h_ref[...]
    hA_ref[...] = jnp.dot(hb, Wr_ref[...], preferred_element_type=jnp.float32)
    hB_ref[...] = jnp.dot(hb, Wc_ref[...], preferred_element_type=jnp.float32)


def _pre_edge(h, Wr, Wc):
    H2 = 2 * N_HIDDEN
    return pl.pallas_call(
        _pre_edge_body,
        grid=(N_NODES // _NB,),
        in_specs=[
            pl.BlockSpec((_NB, N_HIDDEN), lambda i: (i, 0)),
            pl.BlockSpec((N_HIDDEN, H2), lambda i: (0, 0)),
            pl.BlockSpec((N_HIDDEN, H2), lambda i: (0, 0)),
        ],
        out_specs=[
            pl.BlockSpec((_NB, H2), lambda i: (i, 0)),
            pl.BlockSpec((_NB, H2), lambda i: (i, 0)),
        ],
        out_shape=[
            jax.ShapeDtypeStruct((N_NODES, H2), jnp.float32),
            jax.ShapeDtypeStruct((N_NODES, H2), jnp.float32),
        ],
    )(h, Wr, Wc)


def _ln_prelu(u, g_ref, be_ref, a_ref):
    m = jnp.mean(u, axis=1, keepdims=True)
    d = u - m
    v = jnp.mean(d * d, axis=1, keepdims=True)
    u = d * jax.lax.rsqrt(v + 1e-5) * g_ref[...] + be_ref[...]
    return jnp.where(u >= 0, u, a_ref[...] * u)


def _edge_mlp_body(epA_ref, epB_ref, e_ref, We_ref, b1_ref, g_ref, be_ref,
                   a_ref, W2_ref, b2_ref, eoT_ref, en_ref):
    u = (epA_ref[...] + epB_ref[...]
         + jnp.dot(e_ref[...], We_ref[...], preferred_element_type=jnp.float32)
         + b1_ref[...])
    u = _ln_prelu(u, g_ref, be_ref, a_ref)
    eo = (jnp.dot(u, W2_ref[...], preferred_element_type=jnp.float32)
          + b2_ref[...])
    eoT_ref[...] = eo.T
    en_ref[...] = e_ref[...] + eo


def _edge_mlp(epA, epB, e, We, b1, g, be, a, W2, b2):
    H2 = 2 * N_HIDDEN
    wide = pl.BlockSpec((1, H2), lambda i: (0, 0))
    return pl.pallas_call(
        _edge_mlp_body,
        grid=(N_EDGES // _EB,),
        in_specs=[
            pl.BlockSpec((_EB, H2), lambda i: (i, 0)),
            pl.BlockSpec((_EB, H2), lambda i: (i, 0)),
            pl.BlockSpec((_EB, N_HIDDEN), lambda i: (i, 0)),
            pl.BlockSpec((N_HIDDEN, H2), lambda i: (0, 0)),
            wide, wide, wide,
            pl.BlockSpec((1, 1), lambda i: (0, 0)),
            pl.BlockSpec((H2, N_HIDDEN), lambda i: (0, 0)),
            pl.BlockSpec((1, N_HIDDEN), lambda i: (0, 0)),
        ],
        out_specs=[
            pl.BlockSpec((N_HIDDEN, _EB), lambda i: (0, i)),
            pl.BlockSpec((_EB, N_HIDDEN), lambda i: (i, 0)),
        ],
        out_shape=[
            jax.ShapeDtypeStruct((N_HIDDEN, N_EDGES), jnp.float32),
            jax.ShapeDtypeStruct((N_EDGES, N_HIDDEN), jnp.float32),
        ],
    )(epA, epB, e, We, b1.reshape(1, -1), g.reshape(1, -1),
      be.reshape(1, -1), a.reshape(1, 1), W2, b2.reshape(1, -1))


def _agg_mean_body(aggT_ref, degp_ref, o_ref):
    deg = jnp.maximum(jnp.sum(degp_ref[...], axis=0, keepdims=True), 1.0)
    o_ref[...] = (aggT_ref[...] / deg).T


_AB = 2176


def _agg_mean(aggT, degp):
    return pl.pallas_call(
        _agg_mean_body,
        grid=(_NPAD // _AB,),
        in_specs=[
            pl.BlockSpec((N_HIDDEN, _AB), lambda i: (0, i)),
            pl.BlockSpec((NW, _AB), lambda i: (0, i)),
        ],
        out_specs=pl.BlockSpec((_AB, N_HIDDEN), lambda i: (i, 0)),
        out_shape=jax.ShapeDtypeStruct((_NPAD, N_HIDDEN), jnp.float32),
    )(aggT, degp)


def _node_mlp_body(h_ref, agg_ref, Wh_ref, Wa_ref, b1_ref,
                   g_ref, be_ref, a_ref, W2_ref, b2_ref, o_ref):
    hb = h_ref[...]
    u = (jnp.dot(hb, Wh_ref[...], preferred_element_type=jnp.float32)
         + jnp.dot(agg_ref[...], Wa_ref[...],
                   preferred_element_type=jnp.float32)
         + b1_ref[...])
    u = _ln_prelu(u, g_ref, be_ref, a_ref)
    no = (jnp.dot(u, W2_ref[...], preferred_element_type=jnp.float32)
          + b2_ref[...])
    o_ref[...] = hb + no


def _node_mlp(h, agg, Wh, Wa, b1, g, be, a, W2, b2):
    H2 = 2 * N_HIDDEN
    wide = pl.BlockSpec((1, H2), lambda i: (0, 0))
    return pl.pallas_call(
        _node_mlp_body,
        grid=(N_NODES // _NB,),
        in_specs=[
            pl.BlockSpec((_NB, N_HIDDEN), lambda i: (i, 0)),
            pl.BlockSpec((_NB, N_HIDDEN), lambda i: (i, 0)),
            pl.BlockSpec((N_HIDDEN, H2), lambda i: (0, 0)),
            pl.BlockSpec((N_HIDDEN, H2), lambda i: (0, 0)),
            wide, wide, wide,
            pl.BlockSpec((1, 1), lambda i: (0, 0)),
            pl.BlockSpec((H2, N_HIDDEN), lambda i: (0, 0)),
            pl.BlockSpec((1, N_HIDDEN), lambda i: (0, 0)),
        ],
        out_specs=pl.BlockSpec((_NB, N_HIDDEN), lambda i: (i, 0)),
        out_shape=jax.ShapeDtypeStruct((N_NODES, N_HIDDEN), jnp.float32),
    )(h, agg, Wh, Wa, b1.reshape(1, -1), g.reshape(1, -1),
      be.reshape(1, -1), a.reshape(1, 1), W2, b2.reshape(1, -1))


_OB = 2000


def _out_body(h_ref, b3_ref, W1_ref, b1_ref, W2_ref, b2_ref, o_ref, acc_ref):
    i = pl.program_id(0)

    @pl.when(i == 0)
    def _():
        acc_ref[...] = jnp.zeros_like(acc_ref)

    gids = lax.broadcasted_iota(jnp.int32, (N_GRAPHS, _OB), 0)
    onehot = (gids == b3_ref[0]).astype(jnp.float32)
    acc_ref[...] += jnp.dot(onehot, h_ref[...],
                            preferred_element_type=jnp.float32)

    @pl.when(i == (N_NODES // _OB) - 1)
    def _():
        hid = (jnp.dot(acc_ref[...], W1_ref[...],
                       preferred_element_type=jnp.float32) + b1_ref[...])
        hid = jnp.where(hid >= 0, hid, 0.01 * hid)
        o_ref[...] = (jnp.dot(hid, W2_ref[...],
                              preferred_element_type=jnp.float32) + b2_ref[...])


def _out_mlp(h, batch3d, W1, b1, W2, b2):
    nh = W1.shape[1]
    nout = W2.shape[1]
    return pl.pallas_call(
        _out_body,
        grid=(N_NODES // _OB,),
        in_specs=[
            pl.BlockSpec((_OB, N_HIDDEN), lambda i: (i, 0)),
            pl.BlockSpec((1, 1, _OB), lambda i: (i, 0, 0)),
            pl.BlockSpec((N_HIDDEN, nh), lambda i: (0, 0)),
            pl.BlockSpec((1, nh), lambda i: (0, 0)),
            pl.BlockSpec((nh, nout), lambda i: (0, 0)),
            pl.BlockSpec((1, nout), lambda i: (0, 0)),
        ],
        out_specs=pl.BlockSpec((N_GRAPHS, nout), lambda i: (0, 0)),
        out_shape=jax.ShapeDtypeStruct((N_GRAPHS, nout), jnp.float32),
        scratch_shapes=[pltpu.VMEM((N_GRAPHS, N_HIDDEN), jnp.float32)],
    )(h, batch3d, W1, b1.reshape(1, -1), W2, b2.reshape(1, -1))


def kernel(x, edge_vec, edge_index, batch, params):
    p = params
    row = edge_index[0]
    col = edge_index[1]

    h = _enc_nodes(x, p['enc_n_W1'], p['enc_n_b1'], p['enc_n_a'],
                   p['enc_n_W2'], p['enc_n_b2'])
    e = _enc_edges(edge_vec, p['enc_e_W1'], p['enc_e_b1'], p['enc_e_a'],
                   p['enc_e_W2'], p['enc_e_b2'])

    def _fill(shape, val):
        return pl.pallas_call(
            lambda x_ref, o_ref: o_ref.__setitem__(
                (Ellipsis,), jnp.full(o_ref.shape, val, jnp.float32)
                + x_ref[0, 0] * 0.0),
            out_shape=jax.ShapeDtypeStruct(shape, jnp.float32),
        )(x)

    zeros_deg = _fill((_NPAD // 128, 128), 0.0).reshape(_NPAD)
    zeros_sc = _fill((_FPW * _NPAD // 128, 128), 0.0).reshape(_FPW * _NPAD)
    degp = _sc_degree(col, zeros_deg)

    for lp in p['layers']:
        eW1 = lp['e_W1']
        hA, hB = _pre_edge(h, eW1[0:N_HIDDEN], eW1[N_HIDDEN:2 * N_HIDDEN])
        epA, epB = _sc_gather2(hA, hB, row, col)
        eoT, e = _edge_mlp(epA, epB, e, eW1[2 * N_HIDDEN:], lp['e_b1'],
                           lp['e_g'], lp['e_be'], lp['e_a'], lp['e_W2'],
                           lp['e_b2'])
        aggT = _sc_scatter(eoT, col, zeros_sc)
        agg = _agg_mean(aggT, degp)
        nW1 = lp['n_W1']
        h = _node_mlp(h, agg, nW1[0:N_HIDDEN], nW1[N_HIDDEN:],
                      lp['n_b1'], lp['n_g'], lp['n_be'], lp['n_a'],
                      lp['n_W2'], lp['n_b2'])

    batch3d = batch.reshape(N_NODES // _OB, 1, _OB)
    return _out_mlp(h, batch3d, p['out_W1'], p['out_b1'],
                    p['out_W2'], p['out_b2'])
